# baseline (speedup 1.0000x reference)
import functools

import jax
import jax.numpy as jnp
from jax import lax
from jax.experimental import pallas as pl
from jax.experimental.pallas import tpu as pltpu
from jax.experimental.pallas import tpu_sc as plsc

N_NODES = 50000
N_EDGES = 800000
D_IN = 100
D_HID = 128
D_OUT = 2

N_PAD = 50176
N_TAB = 8 * N_PAD
E_PAD = 819200
E_ROWS = E_PAD // 128
BLK = 40
NBUF = 12
LAG = 8
ROWS_FULL = E_ROWS // 16
ROWS_HALF = E_ROWS // 32
SLICE = N_PAD // 16
ZROWS = 784

_MESH = plsc.VectorSubcoreMesh(core_axis_name="c", subcore_axis_name="s",
                               num_cores=2, num_subcores=16)
_SC_PARAMS = pltpu.CompilerParams(use_tc_tiling_on_sc=False)


def _zero_fill(ref, nrows):
    zero = jnp.zeros((16,), jnp.float32)

    def body(i, _):
        ref[i, :] = zero
        return 0

    lax.fori_loop(0, nrows, body, 0)


def _edge_pipeline(nblk, row_base, chunk, adjust, table, acc, src_hbm,
                   dst_hbm, sb, db, adjb, rows, gsems, ssems):
    def block(blk, _):
        @pl.when(blk > 0)
        def _():
            for b in range(NBUF):
                pltpu.make_async_copy(rows[b], acc.at[db.at[0]],
                                      ssems[b]).wait()

        r0 = row_base + blk * BLK
        pltpu.sync_copy(src_hbm.at[pl.ds(r0, BLK)], sb)
        pltpu.sync_copy(dst_hbm.at[pl.ds(r0, BLK)], db)
        gds = [None] * BLK
        sds = [None] * BLK
        for step in range(BLK + LAG):
            if step < BLK:
                r = step
                b = r % NBUF
                if r >= NBUF:
                    sds[r - NBUF].wait()
                if adjust:
                    for g in range(8):
                        v = sb[r, pl.ds(g * 16, 16)]
                        adjb[r, pl.ds(g * 16, 16)] = v + chunk
                    idx_r = adjb.at[r]
                else:
                    idx_r = sb.at[r]
                gds[r] = pltpu.async_copy(table.at[idx_r], rows[b], gsems[b])
            if step >= LAG:
                r = step - LAG
                b = r % NBUF
                gds[r].wait()
                sds[r] = pltpu.async_copy(rows[b], acc.at[db.at[r]],
                                          ssems[b], add=True)
        return 0

    lax.fori_loop(0, nblk, block, 0)
    for b in range(NBUF):
        pltpu.make_async_copy(rows[b], acc.at[db.at[0]], ssems[b]).wait()


def _make_agg_kernel(n_full, split_chunk, split_slot0, adjust):
    @functools.partial(
        pl.kernel,
        out_type=jax.ShapeDtypeStruct((N_PAD, 128), jnp.float32),
        mesh=_MESH,
        scratch_types=dict(
            srcb=pltpu.VMEM((BLK, 128), jnp.int32),
            dstb=pltpu.VMEM((BLK, 128), jnp.int32),
            adjb=pltpu.VMEM((BLK, 128), jnp.int32),
            rows=[pltpu.VMEM((128, 16), jnp.float32) for _ in range(NBUF)],
            zbuf=pltpu.VMEM((ZROWS, 16), jnp.float32),
            bounce=pltpu.VMEM((ZROWS, 16), jnp.float32),
            acc=pltpu.VMEM_SHARED((N_PAD, 16), jnp.float32),
            gsems=[pltpu.SemaphoreType.DMA for _ in range(NBUF)],
            ssems=[pltpu.SemaphoreType.DMA for _ in range(NBUF)],
        ),
        compiler_params=_SC_PARAMS,
    )
    def agg(table_hbm, src_hbm, dst_hbm, out_hbm, *, srcb, dstb,
            adjb, rows, zbuf, bounce, acc, gsems, ssems):
        core = lax.axis_index("c")
        sub = lax.axis_index("s")
        _zero_fill(zbuf, ZROWS)
        my0 = sub * SLICE

        def run_pass(chunk, slot, nblk, row_base):
            for z in range(SLICE // ZROWS):
                pltpu.sync_copy(zbuf, acc.at[pl.ds(my0 + z * ZROWS, ZROWS)])
            plsc.subcore_barrier()
            _edge_pipeline(nblk, row_base, chunk, adjust, table_hbm, acc,
                           src_hbm, dst_hbm, srcb, dstb, adjb, rows,
                           gsems, ssems)
            plsc.subcore_barrier()
            for z in range(SLICE // ZROWS):
                sl = pl.ds(my0 + z * ZROWS, ZROWS)
                pltpu.sync_copy(acc.at[sl],
                                out_hbm.at[sl, pl.ds(slot * 16, 16)])

        if n_full:
            def full_pass(i, _):
                cs = core * n_full + i
                run_pass(cs, cs, ROWS_FULL // BLK, sub * ROWS_FULL)
                return 0

            lax.fori_loop(0, n_full, full_pass, 0)
        run_pass(jnp.int32(split_chunk), split_slot0 + core,
                 ROWS_HALF // BLK, core * (E_ROWS // 2) + sub * ROWS_HALF)

    return agg


_agg_l1 = _make_agg_kernel(n_full=3, split_chunk=6, split_slot0=6,
                           adjust=True)

_agg_l2 = _make_agg_kernel(n_full=0, split_chunk=0, split_slot0=0,
                           adjust=False)


@functools.partial(
    pl.kernel,
    out_type=jax.ShapeDtypeStruct((2, N_PAD), jnp.float32),
    mesh=_MESH,
    scratch_types=dict(
        dstb=pltpu.VMEM((BLK, 128), jnp.int32),
        ones=pltpu.VMEM((128,), jnp.float32),
        buf=pltpu.VMEM((SLICE,), jnp.float32),
        acc=pltpu.VMEM_SHARED((N_PAD,), jnp.float32),
        ssems=[pltpu.SemaphoreType.DMA for _ in range(NBUF)],
    ),
    compiler_params=_SC_PARAMS,
)
def _sc_degree(dst_hbm, out_hbm, *, dstb, ones, buf, acc, ssems):
    core = lax.axis_index("c")
    sub = lax.axis_index("s")
    one = jnp.ones((16,), jnp.float32)
    zero = jnp.zeros((16,), jnp.float32)
    for i in range(8):
        ones[pl.ds(i * 16, 16)] = one

    def zbody(i, _):
        buf[pl.ds(i * 16, 16)] = zero
        return 0

    lax.fori_loop(0, SLICE // 16, zbody, 0)
    my0 = sub * SLICE
    pltpu.sync_copy(buf, acc.at[pl.ds(my0, SLICE)])
    plsc.subcore_barrier()

    wid = core * 16 + sub
    row_base = wid * ROWS_HALF

    def block(blk, _):
        r0 = row_base + blk * BLK
        pltpu.sync_copy(dst_hbm.at[pl.ds(r0, BLK)], dstb)
        sds = [None] * BLK
        for r in range(BLK):
            if r >= NBUF:
                sds[r - NBUF].wait()
            sds[r] = pltpu.async_copy(ones, acc.at[dstb.at[r]],
                                      ssems[r % NBUF], add=True)
        for r in range(BLK - NBUF, BLK):
            sds[r].wait()
        return 0

    lax.fori_loop(0, ROWS_HALF // BLK, block, 0)
    plsc.subcore_barrier()
    sl = pl.ds(my0, SLICE)
    pltpu.sync_copy(acc.at[sl], buf)
    pltpu.sync_copy(buf, out_hbm.at[core].at[sl])


def _tc1_body(degp_ref, x_ref, p_ref):
    deg = degp_ref[0] + degp_ref[1] + 1.0
    dinv = lax.rsqrt(deg)[:, None]
    xb = x_ref[...]
    p_ref[...] = jnp.concatenate(
        [xb[:, :112] * dinv, dinv, xb[:, 113:] * dinv], axis=1)


def _tc2_body(p_ref, q_ref, w1_ref, b1_ref, w2_ref, p2_ref):
    p = p_ref[...]
    q = q_ref[...]
    dinv = p[:, 112:113]
    s = q[:, :96] + p[:, :96]
    c6 = q[:, 96:112] + q[:, 112:128] + p[:, 96:112]
    z = jnp.concatenate([s, c6], axis=1) * dinv
    h = jnp.dot(z, w1_ref[...], preferred_element_type=jnp.float32)
    h = jnp.maximum(h + b1_ref[...], 0.0)
    h2 = jnp.dot(h, w2_ref[...], preferred_element_type=jnp.float32)
    p2 = h2 * dinv
    p2_ref[...] = jnp.concatenate(
        [p2, dinv, jnp.zeros((p2.shape[0], 111), jnp.float32)], axis=1)


def _tc3_body(p2_ref, q2_ref, b2_ref, out_ref):
    q2 = q2_ref[...]
    p2 = p2_ref[...]
    z = (q2[:, 0:16] + q2[:, 16:32] + p2[:, 0:16]) * p2[:, 16:17]
    z = z + b2_ref[...]
    z0 = z[:, 0]
    z1 = z[:, 1]
    m = jnp.maximum(z0, z1)
    lse = m + jnp.log(jnp.exp(z0 - m) + jnp.exp(z1 - m))
    out_ref[:, 0] = z0 - lse
    out_ref[:, 1] = z1 - lse


_TCB = 7168
_GRID = N_PAD // _TCB


def _tc1(degp, x128):
    return pl.pallas_call(
        _tc1_body,
        grid=(_GRID,),
        in_specs=[
            pl.BlockSpec((2, _TCB), lambda i: (0, i)),
            pl.BlockSpec((_TCB, 128), lambda i: (i, 0)),
        ],
        out_specs=pl.BlockSpec((_TCB, 128), lambda i: (i, 0)),
        out_shape=jax.ShapeDtypeStruct((N_PAD, 128), jnp.float32),
    )(degp, x128)


def _tc2(p, q, w1p, b1r, w2p):
    return pl.pallas_call(
        _tc2_body,
        grid=(_GRID,),
        in_specs=[
            pl.BlockSpec((_TCB, 128), lambda i: (i, 0)),
            pl.BlockSpec((_TCB, 128), lambda i: (i, 0)),
            pl.BlockSpec((112, D_HID), lambda i: (0, 0)),
            pl.BlockSpec((1, D_HID), lambda i: (0, 0)),
            pl.BlockSpec((D_HID, 16), lambda i: (0, 0)),
        ],
        out_specs=pl.BlockSpec((_TCB, 128), lambda i: (i, 0)),
        out_shape=jax.ShapeDtypeStruct((N_PAD, 128), jnp.float32),
    )(p, q, w1p, b1r, w2p)


def _tc3(p2, q2, b2r):
    return pl.pallas_call(
        _tc3_body,
        grid=(_GRID,),
        in_specs=[
            pl.BlockSpec((_TCB, 128), lambda i: (i, 0)),
            pl.BlockSpec((_TCB, 128), lambda i: (i, 0)),
            pl.BlockSpec((1, 16), lambda i: (0, 0)),
        ],
        out_specs=pl.BlockSpec((_TCB, D_OUT), lambda i: (i, 0)),
        out_shape=jax.ShapeDtypeStruct((N_NODES, D_OUT), jnp.float32),
    )(p2, q2, b2r)


def kernel(x, edge_index, W1, b1, W2, b2):
    src = edge_index[0]
    dst = edge_index[1]
    n_extra = E_PAD - N_EDGES
    pad_idx = (jnp.arange(n_extra, dtype=jnp.int32) % (N_PAD - N_NODES)
               ) + N_NODES
    src8_p = (jnp.concatenate([src, pad_idx]) * 8).reshape(E_ROWS, 128)
    dst_p = jnp.concatenate([dst, pad_idx]).reshape(E_ROWS, 128)

    x128 = jnp.zeros((N_PAD, 128), jnp.float32).at[:N_NODES, :D_IN].set(x)
    w1p = jnp.zeros((112, D_HID), jnp.float32).at[:D_IN].set(W1)
    b1r = b1.reshape(1, D_HID)
    w2p = jnp.zeros((D_HID, 16), jnp.float32).at[:, :D_OUT].set(W2)
    b2r = jnp.zeros((1, 16), jnp.float32).at[0, :D_OUT].set(b2)

    degp = _sc_degree(dst_p)
    p = _tc1(degp, x128)
    q = _agg_l1(p.reshape(N_TAB, 16), src8_p, dst_p)
    p2 = _tc2(p, q, w1p, b1r, w2p)
    q2 = _agg_l2(p2.reshape(N_TAB, 16), src8_p, dst_p)
    return _tc3(p2, q2, b2r)


# speedup vs baseline: 1.1219x; 1.1219x over previous
import functools

import jax
import jax.numpy as jnp
from jax import lax
from jax.experimental import pallas as pl
from jax.experimental.pallas import tpu as pltpu
from jax.experimental.pallas import tpu_sc as plsc

N_NODES = 50000
N_EDGES = 800000
D_IN = 100
D_HID = 128
D_OUT = 2

N_PAD = 50176
N_TAB = 8 * N_PAD
E_PAD = 819200
E_ROWS = E_PAD // 128
BLK = 40
NBUF = 12
LAG = 8
ROWS_FULL = E_ROWS // 16
ROWS_HALF = E_ROWS // 32
SLICE = N_PAD // 16
ZROWS = 784

_MESH = plsc.VectorSubcoreMesh(core_axis_name="c", subcore_axis_name="s",
                               num_cores=2, num_subcores=16)
_SC_PARAMS = pltpu.CompilerParams(use_tc_tiling_on_sc=False)


def _zero_fill(ref, nrows):
    zero = jnp.zeros((16,), jnp.float32)

    def body(i, _):
        ref[i, :] = zero
        return 0

    lax.fori_loop(0, nrows, body, 0)


def _edge_pipeline(nblk, row_base, chunk, adjust, table, acc, src_hbm,
                   dst_hbm, sb, db, adjb, rows, gsems, ssems):
    def block(blk, _):
        @pl.when(blk > 0)
        def _():
            for b in range(NBUF):
                pltpu.make_async_copy(rows[b], acc.at[db.at[0]],
                                      ssems[b]).wait()

        r0 = row_base + blk * BLK
        pltpu.sync_copy(src_hbm.at[pl.ds(r0, BLK)], sb)
        pltpu.sync_copy(dst_hbm.at[pl.ds(r0, BLK)], db)
        gds = [None] * BLK
        sds = [None] * BLK
        for step in range(BLK + LAG):
            if step < BLK:
                r = step
                b = r % NBUF
                if r >= NBUF:
                    sds[r - NBUF].wait()
                if adjust:
                    for g in range(8):
                        v = sb[r, pl.ds(g * 16, 16)]
                        adjb[r, pl.ds(g * 16, 16)] = v + chunk
                    idx_r = adjb.at[r]
                else:
                    idx_r = sb.at[r]
                gds[r] = pltpu.async_copy(table.at[idx_r], rows[b], gsems[b])
            if step >= LAG:
                r = step - LAG
                b = r % NBUF
                gds[r].wait()
                sds[r] = pltpu.async_copy(rows[b], acc.at[db.at[r]],
                                          ssems[b], add=True)
        return 0

    lax.fori_loop(0, nblk, block, 0)
    for b in range(NBUF):
        pltpu.make_async_copy(rows[b], acc.at[db.at[0]], ssems[b]).wait()


def _make_agg_kernel(n_full, split_chunk, split_slot0, adjust):
    @functools.partial(
        pl.kernel,
        out_type=jax.ShapeDtypeStruct((N_PAD, 128), jnp.float32),
        mesh=_MESH,
        scratch_types=dict(
            srcb=pltpu.VMEM((BLK, 128), jnp.int32),
            dstb=pltpu.VMEM((BLK, 128), jnp.int32),
            adjb=pltpu.VMEM((BLK, 128), jnp.int32),
            rows=[pltpu.VMEM((128, 16), jnp.float32) for _ in range(NBUF)],
            zbuf=pltpu.VMEM((ZROWS, 16), jnp.float32),
            bounce=[pltpu.VMEM((ZROWS, 16), jnp.float32) for _ in range(2)],
            acc=pltpu.VMEM_SHARED((N_PAD, 16), jnp.float32),
            gsems=[pltpu.SemaphoreType.DMA for _ in range(NBUF)],
            ssems=[pltpu.SemaphoreType.DMA for _ in range(NBUF)],
        ),
        compiler_params=_SC_PARAMS,
    )
    def agg(table_hbm, src_hbm, dst_hbm, out_hbm, *, srcb, dstb,
            adjb, rows, zbuf, bounce, acc, gsems, ssems):
        core = lax.axis_index("c")
        sub = lax.axis_index("s")
        _zero_fill(zbuf, ZROWS)
        my0 = sub * SLICE

        def run_pass(chunk, slot, nblk, row_base):
            zds = [pltpu.async_copy(zbuf,
                                    acc.at[pl.ds(my0 + z * ZROWS, ZROWS)],
                                    gsems[z])
                   for z in range(SLICE // ZROWS)]
            for d in zds:
                d.wait()
            plsc.subcore_barrier()
            _edge_pipeline(nblk, row_base, chunk, adjust, table_hbm, acc,
                           src_hbm, dst_hbm, srcb, dstb, adjb, rows,
                           gsems, ssems)
            plsc.subcore_barrier()
            hb = [None, None]
            for z in range(SLICE // ZROWS):
                par = z % 2
                sl = pl.ds(my0 + z * ZROWS, ZROWS)
                if hb[par] is not None:
                    hb[par].wait()
                pltpu.async_copy(acc.at[sl], bounce[par], gsems[par]).wait()
                hb[par] = pltpu.async_copy(
                    bounce[par], out_hbm.at[sl, pl.ds(slot * 16, 16)],
                    ssems[par])
            for d in hb:
                d.wait()

        if n_full:
            def full_pass(i, _):
                cs = core * n_full + i
                run_pass(cs, cs, ROWS_FULL // BLK, sub * ROWS_FULL)
                return 0

            lax.fori_loop(0, n_full, full_pass, 0)
        run_pass(jnp.int32(split_chunk), split_slot0 + core,
                 ROWS_HALF // BLK, core * (E_ROWS // 2) + sub * ROWS_HALF)

    return agg


_agg_l1 = _make_agg_kernel(n_full=3, split_chunk=6, split_slot0=6,
                           adjust=True)

_agg_l2 = _make_agg_kernel(n_full=0, split_chunk=0, split_slot0=0,
                           adjust=False)


@functools.partial(
    pl.kernel,
    out_type=jax.ShapeDtypeStruct((2, N_PAD), jnp.float32),
    mesh=_MESH,
    scratch_types=dict(
        dstb=pltpu.VMEM((BLK, 128), jnp.int32),
        ones=pltpu.VMEM((128,), jnp.float32),
        buf=pltpu.VMEM((SLICE,), jnp.float32),
        acc=pltpu.VMEM_SHARED((N_PAD,), jnp.float32),
        ssems=[pltpu.SemaphoreType.DMA for _ in range(NBUF)],
    ),
    compiler_params=_SC_PARAMS,
)
def _sc_degree(dst_hbm, out_hbm, *, dstb, ones, buf, acc, ssems):
    core = lax.axis_index("c")
    sub = lax.axis_index("s")
    one = jnp.ones((16,), jnp.float32)
    zero = jnp.zeros((16,), jnp.float32)
    for i in range(8):
        ones[pl.ds(i * 16, 16)] = one

    def zbody(i, _):
        buf[pl.ds(i * 16, 16)] = zero
        return 0

    lax.fori_loop(0, SLICE // 16, zbody, 0)
    my0 = sub * SLICE
    pltpu.sync_copy(buf, acc.at[pl.ds(my0, SLICE)])
    plsc.subcore_barrier()

    wid = core * 16 + sub
    row_base = wid * ROWS_HALF

    def block(blk, _):
        r0 = row_base + blk * BLK
        pltpu.sync_copy(dst_hbm.at[pl.ds(r0, BLK)], dstb)
        sds = [None] * BLK
        for r in range(BLK):
            if r >= NBUF:
                sds[r - NBUF].wait()
            sds[r] = pltpu.async_copy(ones, acc.at[dstb.at[r]],
                                      ssems[r % NBUF], add=True)
        for r in range(BLK - NBUF, BLK):
            sds[r].wait()
        return 0

    lax.fori_loop(0, ROWS_HALF // BLK, block, 0)
    plsc.subcore_barrier()
    sl = pl.ds(my0, SLICE)
    pltpu.sync_copy(acc.at[sl], buf)
    pltpu.sync_copy(buf, out_hbm.at[core].at[sl])


def _tc1_body(degp_ref, x_ref, p_ref):
    deg = degp_ref[0] + degp_ref[1] + 1.0
    dinv = lax.rsqrt(deg)[:, None]
    xb = x_ref[...]
    p_ref[...] = jnp.concatenate(
        [xb[:, :112] * dinv, dinv, xb[:, 113:] * dinv], axis=1)


def _tc2_body(p_ref, q_ref, w1_ref, b1_ref, w2_ref, p2_ref):
    p = p_ref[...]
    q = q_ref[...]
    dinv = p[:, 112:113]
    s = q[:, :96] + p[:, :96]
    c6 = q[:, 96:112] + q[:, 112:128] + p[:, 96:112]
    z = jnp.concatenate([s, c6], axis=1) * dinv
    h = jnp.dot(z, w1_ref[...], preferred_element_type=jnp.float32)
    h = jnp.maximum(h + b1_ref[...], 0.0)
    h2 = jnp.dot(h, w2_ref[...], preferred_element_type=jnp.float32)
    p2 = h2 * dinv
    p2_ref[...] = jnp.concatenate(
        [p2, dinv, jnp.zeros((p2.shape[0], 111), jnp.float32)], axis=1)


def _tc3_body(p2_ref, q2_ref, b2_ref, out_ref):
    q2 = q2_ref[...]
    p2 = p2_ref[...]
    z = (q2[:, 0:16] + q2[:, 16:32] + p2[:, 0:16]) * p2[:, 16:17]
    z = z + b2_ref[...]
    z0 = z[:, 0]
    z1 = z[:, 1]
    m = jnp.maximum(z0, z1)
    lse = m + jnp.log(jnp.exp(z0 - m) + jnp.exp(z1 - m))
    out_ref[:, 0] = z0 - lse
    out_ref[:, 1] = z1 - lse


_TCB = 7168
_GRID = N_PAD // _TCB


def _tc1(degp, x128):
    return pl.pallas_call(
        _tc1_body,
        grid=(_GRID,),
        in_specs=[
            pl.BlockSpec((2, _TCB), lambda i: (0, i)),
            pl.BlockSpec((_TCB, 128), lambda i: (i, 0)),
        ],
        out_specs=pl.BlockSpec((_TCB, 128), lambda i: (i, 0)),
        out_shape=jax.ShapeDtypeStruct((N_PAD, 128), jnp.float32),
    )(degp, x128)


def _tc2(p, q, w1p, b1r, w2p):
    return pl.pallas_call(
        _tc2_body,
        grid=(_GRID,),
        in_specs=[
            pl.BlockSpec((_TCB, 128), lambda i: (i, 0)),
            pl.BlockSpec((_TCB, 128), lambda i: (i, 0)),
            pl.BlockSpec((112, D_HID), lambda i: (0, 0)),
            pl.BlockSpec((1, D_HID), lambda i: (0, 0)),
            pl.BlockSpec((D_HID, 16), lambda i: (0, 0)),
        ],
        out_specs=pl.BlockSpec((_TCB, 128), lambda i: (i, 0)),
        out_shape=jax.ShapeDtypeStruct((N_PAD, 128), jnp.float32),
    )(p, q, w1p, b1r, w2p)


def _tc3(p2, q2, b2r):
    return pl.pallas_call(
        _tc3_body,
        grid=(_GRID,),
        in_specs=[
            pl.BlockSpec((_TCB, 128), lambda i: (i, 0)),
            pl.BlockSpec((_TCB, 128), lambda i: (i, 0)),
            pl.BlockSpec((1, 16), lambda i: (0, 0)),
        ],
        out_specs=pl.BlockSpec((_TCB, D_OUT), lambda i: (i, 0)),
        out_shape=jax.ShapeDtypeStruct((N_NODES, D_OUT), jnp.float32),
    )(p2, q2, b2r)


def kernel(x, edge_index, W1, b1, W2, b2):
    src = edge_index[0]
    dst = edge_index[1]
    n_extra = E_PAD - N_EDGES
    pad_idx = (jnp.arange(n_extra, dtype=jnp.int32) % (N_PAD - N_NODES)
               ) + N_NODES
    src8_p = (jnp.concatenate([src, pad_idx]) * 8).reshape(E_ROWS, 128)
    dst_p = jnp.concatenate([dst, pad_idx]).reshape(E_ROWS, 128)

    x128 = jnp.zeros((N_PAD, 128), jnp.float32).at[:N_NODES, :D_IN].set(x)
    w1p = jnp.zeros((112, D_HID), jnp.float32).at[:D_IN].set(W1)
    b1r = b1.reshape(1, D_HID)
    w2p = jnp.zeros((D_HID, 16), jnp.float32).at[:, :D_OUT].set(W2)
    b2r = jnp.zeros((1, 16), jnp.float32).at[0, :D_OUT].set(b2)

    degp = _sc_degree(dst_p)
    p = _tc1(degp, x128)
    q = _agg_l1(p.reshape(N_TAB, 16), src8_p, dst_p)
    p2 = _tc2(p, q, w1p, b1r, w2p)
    q2 = _agg_l2(p2.reshape(N_TAB, 16), src8_p, dst_p)
    return _tc3(p2, q2, b2r)
